# baseline (speedup 1.0000x reference)
import functools

import jax
import jax.numpy as jnp
from jax import lax
from jax.experimental import pallas as pl
from jax.experimental.pallas import tpu as pltpu
from jax.experimental.pallas import tpu_sc as plsc

D = 128
NC = 2
NS = 16
L = 16
NW = NC * NS
RFINE = 32
R_FPR = (D * D) // RFINE
R_PER_W = R_FPR // NW
E_FPR = D // L


def _sc_call(interpret=False):
    return functools.partial(
        pl.kernel,
        out_type=jax.ShapeDtypeStruct((NC, L), jnp.float32),
        mesh=plsc.VectorSubcoreMesh(
            core_axis_name="c", subcore_axis_name="s", num_cores=NC, num_subcores=NS
        ),
        compiler_params=pltpu.CompilerParams(
            needs_layout_passes=False, use_tc_tiling_on_sc=False
        ),
        scratch_types=[
            pltpu.VMEM((L,), jnp.int32),
            pltpu.VMEM((L, L), jnp.float32),
            pltpu.VMEM((R_PER_W, RFINE), jnp.float32),
            pltpu.VMEM((L,), jnp.float32),
            pltpu.VMEM((L,), jnp.float32),
            pltpu.VMEM_SHARED((NS, L), jnp.float32),
            pltpu.VMEM((NS, L), jnp.float32),
            pltpu.SemaphoreType.DMA,
            pltpu.SemaphoreType.DMA,
        ],
        interpret=interpret,
    )


def _holo_body(x_hbm, e_hbm, r_hbm, out_hbm,
               xv, ebuf, rbuf, accv, outv, shared, sbuf, esem, rsem):
    c = lax.axis_index("c")
    t = lax.axis_index("s")
    wid = c * NS + t

    i16 = lax.iota(jnp.int32, L)
    z16 = jnp.zeros((L,), jnp.int32)

    pltpu.sync_copy(x_hbm.at[pl.ds(0, L)], xv)
    pv = plsc.load_gather(xv, [z16 + 2])

    esel = (i16 >= E_FPR).astype(jnp.int32)
    ev = plsc.load_gather(xv, [esel])
    eidx = ev * E_FPR + (i16 % E_FPR)
    ecopy = pltpu.async_copy(e_hbm.at[eidx], ebuf, esem)
    ridx = pv * R_FPR + wid * R_PER_W + i16
    rcopy = pltpu.async_copy(r_hbm.at[ridx], rbuf, rsem)
    ecopy.wait()
    rcopy.wait()

    acc = jnp.zeros((L,), jnp.float32)
    for q in range(4):
        i_sc = wid * 4 + q
        srow = jnp.broadcast_to(i_sc // L, (L,))
        scol = jnp.broadcast_to(i_sc % L, (L,))
        s_i = plsc.load_gather(ebuf, [srow, scol])
        part = jnp.zeros((L,), jnp.float32)
        for u in range(4):
            tt = q * 4 + u
            o_lo = ebuf[E_FPR + 2 * u, :]
            o_hi = ebuf[E_FPR + 2 * u + 1, :]
            part = part + rbuf[tt, pl.ds(0, L)] * o_lo + rbuf[tt, pl.ds(L, L)] * o_hi
        acc = acc + s_i * part

    accv[...] = acc
    pltpu.sync_copy(accv, shared.at[t])
    plsc.subcore_barrier()

    @pl.when(t == 0)
    def _():
        pltpu.sync_copy(shared, sbuf)
        tot = sbuf[0, :]
        for rr in range(1, NS):
            tot = tot + sbuf[rr, :]
        eta = jnp.sum(tot)
        outv[...] = jnp.broadcast_to(eta, (L,))
        pltpu.sync_copy(outv, out_hbm.at[c])


_holo_sc = _sc_call()(_holo_body)


def kernel(x, E_table, R_table):
    xi = x.astype(jnp.int32).reshape(-1)
    e_fine = E_table.reshape(-1, L)
    r_fine = R_table.reshape(-1, RFINE)
    out = _holo_sc(xi, e_fine, r_fine)
    return out[0, 0] + out[1, 0]


# speedup vs baseline: 1.7457x; 1.7457x over previous
import functools

import jax
import jax.numpy as jnp
from jax import lax
from jax.experimental import pallas as pl
from jax.experimental.pallas import tpu as pltpu
from jax.experimental.pallas import tpu_sc as plsc

D = 128
NC = 2
NS = 16
L = 16
NW = NC * NS
R_PER_W = (D * D) // NW


def _sc_call(interpret=False):
    return functools.partial(
        pl.kernel,
        out_type=jax.ShapeDtypeStruct((NC, L), jnp.float32),
        mesh=plsc.VectorSubcoreMesh(
            core_axis_name="c", subcore_axis_name="s", num_cores=NC, num_subcores=NS
        ),
        compiler_params=pltpu.CompilerParams(
            needs_layout_passes=False, use_tc_tiling_on_sc=True
        ),
        scratch_types=[
            pltpu.VMEM((L,), jnp.int32),
            pltpu.VMEM((L, D), jnp.float32),
            pltpu.VMEM((R_PER_W,), jnp.float32),
            pltpu.VMEM((L,), jnp.float32),
            pltpu.VMEM((L,), jnp.float32),
            pltpu.VMEM_SHARED((NS, 128), jnp.float32),
            pltpu.VMEM((NS, 128), jnp.float32),
            pltpu.SemaphoreType.DMA,
            pltpu.SemaphoreType.DMA,
        ],
        interpret=interpret,
    )


def _holo_body(x_hbm, e_hbm, r_hbm, out_hbm,
               xv, ebuf, rbuf, accv, outv, shared, sbuf, esem, rsem):
    c = lax.axis_index("c")
    t = lax.axis_index("s")
    wid = c * NS + t

    i16 = lax.iota(jnp.int32, L)
    z16 = jnp.zeros((L,), jnp.int32)

    pltpu.sync_copy(x_hbm, xv.at[pl.ds(0, 3)])
    pv = plsc.load_gather(xv, [z16 + 2])
    p_sc = lax.reduce_max(pv, (0,))

    eidx = plsc.load_gather(xv, [(i16 != 1).astype(jnp.int32)])
    ecopy = pltpu.async_copy(e_hbm.at[eidx], ebuf, esem)
    rcopy = pltpu.async_copy(
        r_hbm.at[p_sc, pl.ds(wid * R_PER_W, R_PER_W)], rbuf, rsem
    )
    ecopy.wait()
    rcopy.wait()

    one16 = z16 + 1
    acc = jnp.zeros((L,), jnp.float32)
    for q in range(4):
        i_sc = wid * 4 + q
        s_i = plsc.load_gather(ebuf, [one16, jnp.broadcast_to(i_sc, (L,))])
        part = jnp.zeros((L,), jnp.float32)
        for u in range(4):
            tt = q * 4 + u
            o_lo = ebuf[0, pl.ds(u * 32, L)]
            o_hi = ebuf[0, pl.ds(u * 32 + L, L)]
            part = (part + rbuf[pl.ds(tt * 32, L)] * o_lo
                    + rbuf[pl.ds(tt * 32 + L, L)] * o_hi)
        acc = acc + s_i * part

    accv[...] = acc
    pltpu.sync_copy(accv, shared.at[t, pl.ds(0, L)])
    plsc.subcore_barrier()

    @pl.when(t == 0)
    def _():
        pltpu.sync_copy(shared, sbuf)
        tot = sbuf[0, pl.ds(0, L)]
        for rr in range(1, NS):
            tot = tot + sbuf[rr, pl.ds(0, L)]
        eta = jnp.sum(tot)
        outv[...] = jnp.broadcast_to(eta, (L,))
        pltpu.sync_copy(outv, out_hbm.at[c])


_holo_sc = _sc_call()(_holo_body)


def kernel(x, E_table, R_table):
    trip = x[0].astype(jnp.int32)
    out = _holo_sc(trip, E_table, R_table)
    return out[0, 0] + out[1, 0]


# speedup vs baseline: 1.7539x; 1.0047x over previous
import functools

import jax
import jax.numpy as jnp
from jax import lax
from jax.experimental import pallas as pl
from jax.experimental.pallas import tpu as pltpu
from jax.experimental.pallas import tpu_sc as plsc

D = 128
NC = 2
NS = 16
L = 16
NW = NC * NS
R_PER_W = (D * D) // NW


def _sc_call(interpret=False):
    return functools.partial(
        pl.kernel,
        out_type=jax.ShapeDtypeStruct((NC, L), jnp.float32),
        mesh=plsc.VectorSubcoreMesh(
            core_axis_name="c", subcore_axis_name="s", num_cores=NC, num_subcores=NS
        ),
        compiler_params=pltpu.CompilerParams(
            needs_layout_passes=False, use_tc_tiling_on_sc=True
        ),
        scratch_types=[
            pltpu.VMEM((L,), jnp.int32),
            pltpu.VMEM((L, D), jnp.float32),
            pltpu.VMEM((R_PER_W,), jnp.float32),
            pltpu.VMEM((L,), jnp.float32),
            pltpu.VMEM((L,), jnp.float32),
            pltpu.VMEM_SHARED((NS, 128), jnp.float32),
            pltpu.VMEM((NS, 128), jnp.float32),
            pltpu.SemaphoreType.DMA,
            pltpu.SemaphoreType.DMA,
        ],
        interpret=interpret,
    )


def _holo_body(x_hbm, e_hbm, r_hbm, out_hbm,
               xv, ebuf, rbuf, accv, outv, shared, sbuf, esem, rsem):
    c = lax.axis_index("c")
    t = lax.axis_index("s")
    wid = c * NS + t

    i16 = lax.iota(jnp.int32, L)
    z16 = jnp.zeros((L,), jnp.int32)

    with jax.named_scope("xload"):
        pltpu.sync_copy(x_hbm, xv.at[pl.ds(0, 3)])
        pv = plsc.load_gather(xv, [z16 + 2])
        p_sc = lax.reduce_max(pv, (0,))

    with jax.named_scope("edma"):
        eidx = plsc.load_gather(xv, [(i16 != 1).astype(jnp.int32)])
        ecopy = pltpu.async_copy(e_hbm.at[eidx], ebuf, esem)
        ecopy.wait()
    with jax.named_scope("rdma"):
        rcopy = pltpu.async_copy(
            r_hbm.at[p_sc, pl.ds(wid * R_PER_W, R_PER_W)], rbuf, rsem
        )
        rcopy.wait()

    one16 = z16 + 1
    acc = jnp.zeros((L,), jnp.float32)
    for q in range(4):
        i_sc = wid * 4 + q
        s_i = plsc.load_gather(ebuf, [one16, jnp.broadcast_to(i_sc, (L,))])
        part = jnp.zeros((L,), jnp.float32)
        for u in range(4):
            tt = q * 4 + u
            o_lo = ebuf[0, pl.ds(u * 32, L)]
            o_hi = ebuf[0, pl.ds(u * 32 + L, L)]
            part = (part + rbuf[pl.ds(tt * 32, L)] * o_lo
                    + rbuf[pl.ds(tt * 32 + L, L)] * o_hi)
        acc = acc + s_i * part

    with jax.named_scope("reduce"):
        accv[...] = acc
        pltpu.sync_copy(accv, shared.at[t, pl.ds(0, L)])
        plsc.subcore_barrier()

    @pl.when(t == 0)
    def _():
        pltpu.sync_copy(shared, sbuf)
        tot = sbuf[0, pl.ds(0, L)]
        for rr in range(1, NS):
            tot = tot + sbuf[rr, pl.ds(0, L)]
        eta = jnp.sum(tot)
        outv[...] = jnp.broadcast_to(eta, (L,))
        pltpu.sync_copy(outv, out_hbm.at[c])


_holo_sc = _sc_call()(_holo_body)


def kernel(x, E_table, R_table):
    trip = x[0].astype(jnp.int32)
    out = _holo_sc(trip, E_table, R_table)
    return out[0, 0] + out[1, 0]


# speedup vs baseline: 2.9734x; 1.6953x over previous
import functools

import jax
import jax.numpy as jnp
from jax import lax
from jax.experimental import pallas as pl
from jax.experimental.pallas import tpu as pltpu
from jax.experimental.pallas import tpu_sc as plsc

D = 128
NC = 2
NS = 16
L = 16
NW = NC * NS
R_PER_W = (D * D) // NW


def _sc_call(interpret=False):
    return functools.partial(
        pl.kernel,
        out_type=jax.ShapeDtypeStruct((NC, L), jnp.float32),
        mesh=plsc.VectorSubcoreMesh(
            core_axis_name="c", subcore_axis_name="s", num_cores=NC, num_subcores=NS
        ),
        compiler_params=pltpu.CompilerParams(
            needs_layout_passes=False, use_tc_tiling_on_sc=True
        ),
        scratch_types=[
            pltpu.VMEM((L,), jnp.int32),
            pltpu.VMEM((2,), jnp.int32),
            pltpu.VMEM((2, D), jnp.float32),
            pltpu.VMEM((R_PER_W,), jnp.float32),
            pltpu.VMEM((L,), jnp.float32),
            pltpu.VMEM((L,), jnp.float32),
            pltpu.VMEM_SHARED((NS, 128), jnp.float32),
            pltpu.VMEM((NS, 128), jnp.float32),
            pltpu.SemaphoreType.DMA,
            pltpu.SemaphoreType.DMA,
        ],
        interpret=interpret,
    )


def _holo_body(x_hbm, e_hbm, r_hbm, out_hbm,
               xv, ei2, ebuf, rbuf, accv, outv, shared, sbuf, esem, rsem):
    c = lax.axis_index("c")
    t = lax.axis_index("s")
    wid = c * NS + t

    i16 = lax.iota(jnp.int32, L)
    z16 = jnp.zeros((L,), jnp.int32)

    with jax.named_scope("xload"):
        pltpu.sync_copy(x_hbm, xv.at[pl.ds(0, 3)])
        pv = plsc.load_gather(xv, [z16 + 2])
        p_sc = lax.reduce_max(pv, (0,))

    with jax.named_scope("edma"):
        eidx = plsc.load_gather(xv, [(i16 != 1).astype(jnp.int32)])
        plsc.store_scatter(ei2, [i16], eidx, mask=i16 < 2)
        ecopy = pltpu.async_copy(e_hbm.at[ei2], ebuf, esem)
        ecopy.wait()
    with jax.named_scope("rdma"):
        rcopy = pltpu.async_copy(
            r_hbm.at[p_sc, pl.ds(wid * R_PER_W, R_PER_W)], rbuf, rsem
        )
        rcopy.wait()

    one16 = z16 + 1
    acc = jnp.zeros((L,), jnp.float32)
    for q in range(4):
        i_sc = wid * 4 + q
        s_i = plsc.load_gather(ebuf, [one16, jnp.broadcast_to(i_sc, (L,))])
        part = jnp.zeros((L,), jnp.float32)
        for u in range(4):
            tt = q * 4 + u
            o_lo = ebuf[0, pl.ds(u * 32, L)]
            o_hi = ebuf[0, pl.ds(u * 32 + L, L)]
            part = (part + rbuf[pl.ds(tt * 32, L)] * o_lo
                    + rbuf[pl.ds(tt * 32 + L, L)] * o_hi)
        acc = acc + s_i * part

    with jax.named_scope("reduce"):
        accv[...] = acc
        pltpu.sync_copy(accv, shared.at[t, pl.ds(0, L)])
        plsc.subcore_barrier()

    @pl.when(t == 0)
    def _():
        pltpu.sync_copy(shared, sbuf)
        tot = sbuf[0, pl.ds(0, L)]
        for rr in range(1, NS):
            tot = tot + sbuf[rr, pl.ds(0, L)]
        eta = jnp.sum(tot)
        outv[...] = jnp.broadcast_to(eta, (L,))
        pltpu.sync_copy(outv, out_hbm.at[c])


_holo_sc = _sc_call()(_holo_body)


def kernel(x, E_table, R_table):
    trip = x[0].astype(jnp.int32)
    out = _holo_sc(trip, E_table, R_table)
    return out[0, 0] + out[1, 0]


# speedup vs baseline: 3.6365x; 1.2230x over previous
import functools

import jax
import jax.numpy as jnp
from jax import lax
from jax.experimental import pallas as pl
from jax.experimental.pallas import tpu as pltpu
from jax.experimental.pallas import tpu_sc as plsc

D = 128
NS = 16
L = 16
R_PER_W = (D * D) // NS
QN = D // NS


_sc_call = functools.partial(
    pl.kernel,
    out_type=jax.ShapeDtypeStruct((1,), jnp.float32),
    mesh=plsc.VectorSubcoreMesh(
        core_axis_name="c", subcore_axis_name="s", num_cores=1, num_subcores=NS
    ),
    compiler_params=pltpu.CompilerParams(
        needs_layout_passes=False, use_tc_tiling_on_sc=True
    ),
    scratch_types=[
        pltpu.VMEM((L,), jnp.int32),
        pltpu.VMEM((2,), jnp.int32),
        pltpu.VMEM((2, D), jnp.float32),
        pltpu.VMEM((R_PER_W,), jnp.float32),
        pltpu.VMEM((L,), jnp.float32),
        pltpu.VMEM((L,), jnp.float32),
        pltpu.VMEM_SHARED((NS, 128), jnp.float32),
        pltpu.VMEM((NS, 128), jnp.float32),
        pltpu.SemaphoreType.DMA,
        pltpu.SemaphoreType.DMA,
    ],
)


def _holo_body(x_hbm, e_hbm, r_hbm, out_hbm,
               xv, ei2, ebuf, rbuf, accv, outv, shared, sbuf, esem, rsem):
    t = lax.axis_index("s")

    i16 = lax.iota(jnp.int32, L)
    z16 = jnp.zeros((L,), jnp.int32)

    pltpu.sync_copy(x_hbm, xv.at[pl.ds(0, 3)])
    pv = plsc.load_gather(xv, [z16 + 2])
    p_sc = lax.reduce_max(pv, (0,))

    eidx = plsc.load_gather(xv, [(i16 != 1).astype(jnp.int32)])
    plsc.store_scatter(ei2, [i16], eidx, mask=i16 < 2)
    ecopy = pltpu.async_copy(e_hbm.at[ei2], ebuf, esem)
    rcopy = pltpu.async_copy(
        r_hbm.at[p_sc, pl.ds(t * R_PER_W, R_PER_W)], rbuf, rsem
    )
    ecopy.wait()
    rcopy.wait()

    one16 = z16 + 1
    acc = jnp.zeros((L,), jnp.float32)
    for q in range(QN):
        i_sc = t * QN + q
        s_i = plsc.load_gather(ebuf, [one16, jnp.broadcast_to(i_sc, (L,))])
        part = jnp.zeros((L,), jnp.float32)
        for u in range(4):
            tt = q * 4 + u
            o_lo = ebuf[0, pl.ds(u * 32, L)]
            o_hi = ebuf[0, pl.ds(u * 32 + L, L)]
            part = (part + rbuf[pl.ds(tt * 32, L)] * o_lo
                    + rbuf[pl.ds(tt * 32 + L, L)] * o_hi)
        acc = acc + s_i * part

    accv[...] = acc
    pltpu.sync_copy(accv, shared.at[t, pl.ds(0, L)])
    plsc.subcore_barrier()

    @pl.when(t == 0)
    def _():
        pltpu.sync_copy(shared, sbuf)
        tot = sbuf[0, pl.ds(0, L)]
        for rr in range(1, NS):
            tot = tot + sbuf[rr, pl.ds(0, L)]
        eta = jnp.sum(tot)
        outv[...] = jnp.broadcast_to(eta, (L,))
        pltpu.sync_copy(outv.at[pl.ds(0, 1)], out_hbm)


_holo_sc = _sc_call(_holo_body)


def kernel(x, E_table, R_table):
    trip = x[0].astype(jnp.int32)
    return _holo_sc(trip, E_table, R_table).reshape(())
